# baseline (speedup 1.0000x reference)
import functools

import jax
import jax.numpy as jnp
from jax import lax
from jax.experimental import pallas as pl
from jax.experimental.pallas import tpu as pltpu
from jax.experimental.pallas import tpu_sc as plsc

_PI = 3.141592653589793

NUM_ENT = 1000000
NUM_REL = 1000
D = 64
B = 16384
NC = 2
NS = 16
NW = NC * NS
BPW = B // NW
CHUNK = 128
NCHUNK = BPW // CHUNK
NGRP = CHUNK // 16


_TBLK = 4096


def _tr_body(in_ref, eye_ref, out_ref):
    out_ref[...] = lax.dot_general(
        in_ref[...], eye_ref[...],
        (((0,), (0,)), ((), ())),
        preferred_element_type=jnp.float32)


def _to_row_major(ent_t):
    grid = (NUM_ENT + _TBLK - 1) // _TBLK
    return pl.pallas_call(
        _tr_body,
        grid=(grid,),
        in_specs=[
            pl.BlockSpec((D, _TBLK), lambda i: (0, i)),
            pl.BlockSpec((D, D), lambda i: (0, 0)),
        ],
        out_specs=pl.BlockSpec((_TBLK, D), lambda i: (i, 0)),
        out_shape=jax.ShapeDtypeStruct((NUM_ENT, D), jnp.float32),
    )(ent_t, jnp.eye(D, dtype=jnp.float32))


def _cos_body(rel_ref, out_ref):
    out_ref[...] = jnp.cos(rel_ref[...] * jnp.float32(_PI))


def _cos_table(rel_table):
    return pl.pallas_call(
        _cos_body,
        out_shape=jax.ShapeDtypeStruct((NUM_REL, D), jnp.float32),
    )(rel_table)


def _scal(v, j):
    return lax.squeeze(lax.slice(v, (j,), (j + 1,)), (0,))


_GATHER_DNUMS = lax.GatherDimensionNumbers(
    offset_dims=(), collapsed_slice_dims=(0,), start_index_map=(0,))


def _rotate16(v, sh):
    idx = (lax.iota(jnp.int32, 16) + sh) & 15
    return lax.gather(v, idx[:, None], _GATHER_DNUMS, (1,),
                      indices_are_sorted=False, unique_indices=False,
                      mode=lax.GatherScatterMode.PROMISE_IN_BOUNDS)


def _hsum16(v):
    for sh in (8, 4, 2, 1):
        v = v + _rotate16(v, sh)
    return v


def _sqrt16(x):
    i = lax.bitcast_convert_type(x, jnp.int32)
    y = lax.bitcast_convert_type(jnp.int32(0x5F3759DF) - (i >> 1), jnp.float32)
    xh = 0.5 * x
    for _ in range(3):
        y = y * (1.5 - xh * y * y)
    return x * y


_MESH = plsc.VectorSubcoreMesh(core_axis_name="c", subcore_axis_name="s")


@functools.partial(
    pl.kernel,
    mesh=_MESH,
    compiler_params=pltpu.CompilerParams(use_tc_tiling_on_sc=True),
    out_type=jax.ShapeDtypeStruct((B,), jnp.float32),
    scratch_types=[
        pltpu.VMEM((NCHUNK, CHUNK), jnp.int32),
        pltpu.VMEM((NCHUNK, CHUNK), jnp.int32),
        pltpu.VMEM((NCHUNK, CHUNK), jnp.int32),
        pltpu.VMEM((CHUNK, D), jnp.float32),
        pltpu.VMEM((CHUNK, D), jnp.float32),
        pltpu.VMEM((CHUNK, D), jnp.float32),
        pltpu.VMEM((BPW,), jnp.float32),
        pltpu.SemaphoreType.DMA,
    ],
)
def _sc_score(ent_hbm, cos_hbm, hidx_hbm, ridx_hbm, tidx_hbm, out_hbm,
              hidx_v, ridx_v, tidx_v, h_rows, c_rows, t_rows, out_v, sem):
    wid = lax.axis_index("s") * NC + lax.axis_index("c")
    base = pl.multiple_of(wid * BPW, BPW)

    pltpu.sync_copy(hidx_hbm.at[wid], hidx_v)
    pltpu.sync_copy(ridx_hbm.at[wid], ridx_v)
    pltpu.sync_copy(tidx_hbm.at[wid], tidx_v)

    lanes = lax.iota(jnp.int32, 16)

    def drain_group(row0):
        sl = pl.ds(row0, 16)
        pltpu.make_async_copy(cos_hbm.at[pl.ds(0, 16)], h_rows.at[sl], sem).wait()
        pltpu.make_async_copy(cos_hbm.at[pl.ds(0, 16)], t_rows.at[sl], sem).wait()
        pltpu.make_async_copy(cos_hbm.at[pl.ds(0, 16)], c_rows.at[sl], sem).wait()

    for k in range(NCHUNK):
        def fetch(g, carry):
            row0 = g * 16
            sl = pl.ds(row0, 16)
            hv = hidx_v[k, sl]
            tv = tidx_v[k, sl]
            rv = ridx_v[k, sl]
            for rr in range(16):
                pltpu.async_copy(ent_hbm.at[_scal(hv, rr)],
                                 h_rows.at[row0 + rr], sem)
                pltpu.async_copy(ent_hbm.at[_scal(tv, rr)],
                                 t_rows.at[row0 + rr], sem)
                pltpu.async_copy(cos_hbm.at[_scal(rv, rr)],
                                 c_rows.at[row0 + rr], sem)

            @pl.when(g > 0)
            def _():
                drain_group(row0 - 16)

            return carry

        lax.fori_loop(0, NGRP, fetch, 0)
        drain_group(CHUNK - 16)

        def group(g, carry):
            row0 = g * 16
            ov = jnp.zeros((16,), jnp.float32)
            for rr in range(16):
                i = row0 + rr
                acc = jnp.zeros((16,), jnp.float32)
                for j in range(D // 16):
                    sl = pl.ds(j * 16, 16)
                    hv = h_rows[i, sl]
                    tv = t_rows[i, sl]
                    cv = c_rows[i, sl]
                    x = hv * hv + tv * tv - 2.0 * (hv * tv) * cv
                    acc = acc + _sqrt16(jnp.maximum(x, 0.0))
                ov = jnp.where(lanes == rr, -_hsum16(acc), ov)
            out_v[pl.ds(k * CHUNK + row0, 16)] = ov
            return carry

        lax.fori_loop(0, NGRP, group, 0)

    pltpu.sync_copy(out_v, out_hbm.at[pl.ds(base, BPW)])


def kernel(h_idx, r_idx, t_idx, ent_table, rel_table):
    cos_table = _cos_table(rel_table)
    shp = (NW, NCHUNK, CHUNK)
    h3 = h_idx.astype(jnp.int32).reshape(shp)
    r3 = r_idx.astype(jnp.int32).reshape(shp)
    t3 = t_idx.astype(jnp.int32).reshape(shp)
    ent_rm = _to_row_major(ent_table.T)
    return _sc_score(ent_rm, cos_table, h3, r3, t3)


# speedup vs baseline: 1.3358x; 1.3358x over previous
import functools

import jax
import jax.numpy as jnp
from jax import lax
from jax.experimental import pallas as pl
from jax.experimental.pallas import tpu as pltpu
from jax.experimental.pallas import tpu_sc as plsc

_PI = 3.141592653589793

NUM_ENT = 1000000
NUM_REL = 1000
D = 64
B = 16384
NC = 2
NS = 16
NW = NC * NS
BPW = B // NW
CHUNK = 128
NCHUNK = BPW // CHUNK
NGRP = CHUNK // 16


_TBLK = 16384


def _tr_body(in_ref, eye_ref, out_ref):
    out_ref[...] = lax.dot_general(
        in_ref[...], eye_ref[...],
        (((0,), (0,)), ((), ())),
        preferred_element_type=jnp.float32)


def _to_row_major(ent_t):
    grid = (NUM_ENT + _TBLK - 1) // _TBLK
    return pl.pallas_call(
        _tr_body,
        grid=(grid,),
        in_specs=[
            pl.BlockSpec((D, _TBLK), lambda i: (0, i)),
            pl.BlockSpec((D, D), lambda i: (0, 0)),
        ],
        out_specs=pl.BlockSpec((_TBLK, D), lambda i: (i, 0)),
        out_shape=jax.ShapeDtypeStruct((NUM_ENT, D), jnp.float32),
    )(ent_t, jnp.eye(D, dtype=jnp.float32))


def _cos_body(rel_ref, out_ref):
    out_ref[...] = jnp.cos(rel_ref[...] * jnp.float32(_PI))


def _cos_table(rel_table):
    return pl.pallas_call(
        _cos_body,
        out_shape=jax.ShapeDtypeStruct((NUM_REL, D), jnp.float32),
    )(rel_table)


def _scal(v, j):
    return lax.squeeze(lax.slice(v, (j,), (j + 1,)), (0,))


_GATHER_DNUMS = lax.GatherDimensionNumbers(
    offset_dims=(), collapsed_slice_dims=(0,), start_index_map=(0,))


def _rotate16(v, sh):
    idx = (lax.iota(jnp.int32, 16) + sh) & 15
    return lax.gather(v, idx[:, None], _GATHER_DNUMS, (1,),
                      indices_are_sorted=False, unique_indices=False,
                      mode=lax.GatherScatterMode.PROMISE_IN_BOUNDS)


def _hsum16(v):
    for sh in (8, 4, 2, 1):
        v = v + _rotate16(v, sh)
    return v


def _sqrt16(x):
    i = lax.bitcast_convert_type(x, jnp.int32)
    y = lax.bitcast_convert_type(jnp.int32(0x5F3759DF) - (i >> 1), jnp.float32)
    xh = 0.5 * x
    for _ in range(3):
        y = y * (1.5 - xh * y * y)
    return x * y


_MESH = plsc.VectorSubcoreMesh(core_axis_name="c", subcore_axis_name="s")


@functools.partial(
    pl.kernel,
    mesh=_MESH,
    compiler_params=pltpu.CompilerParams(use_tc_tiling_on_sc=True),
    out_type=jax.ShapeDtypeStruct((B,), jnp.float32),
    scratch_types=[
        pltpu.VMEM((NCHUNK, CHUNK), jnp.int32),
        pltpu.VMEM((NCHUNK, CHUNK), jnp.int32),
        pltpu.VMEM((NCHUNK, CHUNK), jnp.int32),
        pltpu.VMEM((CHUNK, D), jnp.float32),
        pltpu.VMEM((CHUNK, D), jnp.float32),
        pltpu.VMEM((CHUNK, D), jnp.float32),
        pltpu.VMEM((BPW,), jnp.float32),
        pltpu.SemaphoreType.DMA,
    ],
)
def _sc_score(ent_hbm, cos_hbm, hidx_hbm, ridx_hbm, tidx_hbm, out_hbm,
              hidx_v, ridx_v, tidx_v, h_rows, c_rows, t_rows, out_v, sem):
    wid = lax.axis_index("s") * NC + lax.axis_index("c")
    base = pl.multiple_of(wid * BPW, BPW)

    pltpu.sync_copy(hidx_hbm.at[wid], hidx_v)
    pltpu.sync_copy(ridx_hbm.at[wid], ridx_v)
    pltpu.sync_copy(tidx_hbm.at[wid], tidx_v)

    lanes = lax.iota(jnp.int32, 16)

    def drain_group(row0):
        sl = pl.ds(row0, 16)
        pltpu.make_async_copy(cos_hbm.at[pl.ds(0, 16)], h_rows.at[sl], sem).wait()
        pltpu.make_async_copy(cos_hbm.at[pl.ds(0, 16)], t_rows.at[sl], sem).wait()
        pltpu.make_async_copy(cos_hbm.at[pl.ds(0, 16)], c_rows.at[sl], sem).wait()

    for k in range(NCHUNK):
        def fetch(g, carry):
            row0 = g * 16
            sl = pl.ds(row0, 16)
            hv = hidx_v[k, sl]
            tv = tidx_v[k, sl]
            rv = ridx_v[k, sl]
            for rr in range(16):
                pltpu.async_copy(ent_hbm.at[_scal(hv, rr)],
                                 h_rows.at[row0 + rr], sem)
                pltpu.async_copy(ent_hbm.at[_scal(tv, rr)],
                                 t_rows.at[row0 + rr], sem)
                pltpu.async_copy(cos_hbm.at[_scal(rv, rr)],
                                 c_rows.at[row0 + rr], sem)

            @pl.when(g > 0)
            def _():
                drain_group(row0 - 16)

            return carry

        lax.fori_loop(0, NGRP, fetch, 0)
        drain_group(CHUNK - 16)

        def group(g, carry):
            row0 = g * 16
            ov = jnp.zeros((16,), jnp.float32)
            for rr in range(16):
                i = row0 + rr
                acc = jnp.zeros((16,), jnp.float32)
                for j in range(D // 16):
                    sl = pl.ds(j * 16, 16)
                    hv = h_rows[i, sl]
                    tv = t_rows[i, sl]
                    cv = c_rows[i, sl]
                    x = hv * hv + tv * tv - 2.0 * (hv * tv) * cv
                    acc = acc + _sqrt16(jnp.maximum(x, 0.0))
                ov = jnp.where(lanes == rr, -_hsum16(acc), ov)
            out_v[pl.ds(k * CHUNK + row0, 16)] = ov
            return carry

        lax.fori_loop(0, NGRP, group, 0)

    pltpu.sync_copy(out_v, out_hbm.at[pl.ds(base, BPW)])


def kernel(h_idx, r_idx, t_idx, ent_table, rel_table):
    cos_table = _cos_table(rel_table)
    shp = (NW, NCHUNK, CHUNK)
    h3 = h_idx.astype(jnp.int32).reshape(shp)
    r3 = r_idx.astype(jnp.int32).reshape(shp)
    t3 = t_idx.astype(jnp.int32).reshape(shp)
    ent_rm = _to_row_major(ent_table.T)
    return _sc_score(ent_rm, cos_table, h3, r3, t3)


# speedup vs baseline: 1.3656x; 1.0223x over previous
import functools

import jax
import jax.numpy as jnp
from jax import lax
from jax.experimental import pallas as pl
from jax.experimental.pallas import tpu as pltpu
from jax.experimental.pallas import tpu_sc as plsc

_PI = 3.141592653589793

NUM_ENT = 1000000
NUM_REL = 1000
D = 64
B = 16384
NC = 2
NS = 16
NW = NC * NS
BPW = B // NW
CHUNK = 128
NCHUNK = BPW // CHUNK
NGRP = CHUNK // 16


_TBLK = 32768


def _tr_body(in_ref, eye_ref, out_ref):
    out_ref[...] = lax.dot_general(
        in_ref[...], eye_ref[...],
        (((0,), (0,)), ((), ())),
        preferred_element_type=jnp.float32)


def _to_row_major(ent_t):
    grid = (NUM_ENT + _TBLK - 1) // _TBLK
    return pl.pallas_call(
        _tr_body,
        grid=(grid,),
        in_specs=[
            pl.BlockSpec((D, _TBLK), lambda i: (0, i)),
            pl.BlockSpec((D, D), lambda i: (0, 0)),
        ],
        out_specs=pl.BlockSpec((_TBLK, D), lambda i: (i, 0)),
        out_shape=jax.ShapeDtypeStruct((NUM_ENT, D), jnp.float32),
    )(ent_t, jnp.eye(D, dtype=jnp.float32))


def _cos_body(rel_ref, out_ref):
    out_ref[...] = jnp.cos(rel_ref[...] * jnp.float32(_PI))


def _cos_table(rel_table):
    return pl.pallas_call(
        _cos_body,
        out_shape=jax.ShapeDtypeStruct((NUM_REL, D), jnp.float32),
    )(rel_table)


def _scal(v, j):
    return lax.squeeze(lax.slice(v, (j,), (j + 1,)), (0,))


_GATHER_DNUMS = lax.GatherDimensionNumbers(
    offset_dims=(), collapsed_slice_dims=(0,), start_index_map=(0,))


def _rotate16(v, sh):
    idx = (lax.iota(jnp.int32, 16) + sh) & 15
    return lax.gather(v, idx[:, None], _GATHER_DNUMS, (1,),
                      indices_are_sorted=False, unique_indices=False,
                      mode=lax.GatherScatterMode.PROMISE_IN_BOUNDS)


def _hsum16(v):
    for sh in (8, 4, 2, 1):
        v = v + _rotate16(v, sh)
    return v


def _sqrt16(x):
    i = lax.bitcast_convert_type(x, jnp.int32)
    y = lax.bitcast_convert_type(jnp.int32(0x5F3759DF) - (i >> 1), jnp.float32)
    xh = 0.5 * x
    for _ in range(3):
        y = y * (1.5 - xh * y * y)
    return x * y


_MESH = plsc.VectorSubcoreMesh(core_axis_name="c", subcore_axis_name="s")


@functools.partial(
    pl.kernel,
    mesh=_MESH,
    compiler_params=pltpu.CompilerParams(use_tc_tiling_on_sc=True),
    out_type=jax.ShapeDtypeStruct((B,), jnp.float32),
    scratch_types=[
        pltpu.VMEM((NCHUNK, CHUNK), jnp.int32),
        pltpu.VMEM((NCHUNK, CHUNK), jnp.int32),
        pltpu.VMEM((NCHUNK, CHUNK), jnp.int32),
        pltpu.VMEM((CHUNK, D), jnp.float32),
        pltpu.VMEM((CHUNK, D), jnp.float32),
        pltpu.VMEM((CHUNK, D), jnp.float32),
        pltpu.VMEM((BPW,), jnp.float32),
        pltpu.SemaphoreType.DMA,
    ],
)
def _sc_score(ent_hbm, cos_hbm, hidx_hbm, ridx_hbm, tidx_hbm, out_hbm,
              hidx_v, ridx_v, tidx_v, h_rows, c_rows, t_rows, out_v, sem):
    wid = lax.axis_index("s") * NC + lax.axis_index("c")
    base = pl.multiple_of(wid * BPW, BPW)

    pltpu.sync_copy(hidx_hbm.at[wid], hidx_v)
    pltpu.sync_copy(ridx_hbm.at[wid], ridx_v)
    pltpu.sync_copy(tidx_hbm.at[wid], tidx_v)

    lanes = lax.iota(jnp.int32, 16)

    def drain_group(row0):
        sl = pl.ds(row0, 16)
        pltpu.make_async_copy(cos_hbm.at[pl.ds(0, 16)], h_rows.at[sl], sem).wait()
        pltpu.make_async_copy(cos_hbm.at[pl.ds(0, 16)], t_rows.at[sl], sem).wait()
        pltpu.make_async_copy(cos_hbm.at[pl.ds(0, 16)], c_rows.at[sl], sem).wait()

    for k in range(NCHUNK):
        def fetch(g, carry):
            row0 = g * 16
            sl = pl.ds(row0, 16)
            hv = hidx_v[k, sl]
            tv = tidx_v[k, sl]
            rv = ridx_v[k, sl]
            for rr in range(16):
                pltpu.async_copy(ent_hbm.at[_scal(hv, rr)],
                                 h_rows.at[row0 + rr], sem)
                pltpu.async_copy(ent_hbm.at[_scal(tv, rr)],
                                 t_rows.at[row0 + rr], sem)
                pltpu.async_copy(cos_hbm.at[_scal(rv, rr)],
                                 c_rows.at[row0 + rr], sem)

            @pl.when(g > 0)
            def _():
                drain_group(row0 - 16)

            return carry

        lax.fori_loop(0, NGRP, fetch, 0)
        drain_group(CHUNK - 16)

        def group(g, carry):
            row0 = g * 16
            ov = jnp.zeros((16,), jnp.float32)
            for rr in range(16):
                i = row0 + rr
                acc = jnp.zeros((16,), jnp.float32)
                for j in range(D // 16):
                    sl = pl.ds(j * 16, 16)
                    hv = h_rows[i, sl]
                    tv = t_rows[i, sl]
                    cv = c_rows[i, sl]
                    x = hv * hv + tv * tv - 2.0 * (hv * tv) * cv
                    acc = acc + _sqrt16(jnp.maximum(x, 0.0))
                ov = jnp.where(lanes == rr, -_hsum16(acc), ov)
            out_v[pl.ds(k * CHUNK + row0, 16)] = ov
            return carry

        lax.fori_loop(0, NGRP, group, 0)

    pltpu.sync_copy(out_v, out_hbm.at[pl.ds(base, BPW)])


def kernel(h_idx, r_idx, t_idx, ent_table, rel_table):
    cos_table = _cos_table(rel_table)
    shp = (NW, NCHUNK, CHUNK)
    h3 = h_idx.astype(jnp.int32).reshape(shp)
    r3 = r_idx.astype(jnp.int32).reshape(shp)
    t3 = t_idx.astype(jnp.int32).reshape(shp)
    ent_rm = _to_row_major(ent_table.T)
    return _sc_score(ent_rm, cos_table, h3, r3, t3)
